# Initial kernel scaffold; baseline (speedup 1.0000x reference)
import jax
import jax.numpy as jnp
from jax.experimental import pallas as pl


def kernel(x0, x1, c0, c1, W_hg0, b_hg0, W_fc1, b_fc1, W_hg1, b_hg1, W_last, b_last):
    raise NotImplementedError("write your pallas kernel here")


# speedup vs baseline: 17.5535x; 17.5535x over previous
import jax
import jax.numpy as jnp
from jax.experimental import pallas as pl

_K = 16
_NEG_SLOPE = 0.01
_BM = 256


def _dot(a, b, dims):
    return jax.lax.dot_general(a, b, (dims, ((), ())),
                               preferred_element_type=jnp.float32)


def _dist_tile(xr, xf, sq_col, sq_row):
    inner = _dot(xr, xf, ((1,), (1,)))
    return (-2.0 * inner + sq_col) + sq_row


def _edge_kernel(xr_ref, xf_ref, sqc_ref, sqr_ref, w_ref, b_ref,
                 edge_ref, t_ref):
    xr = xr_ref[0]
    xf = xf_ref[0]
    dis = _dist_tile(xr, xf, sqc_ref[0], sqr_ref[0])
    work = dis
    t = None
    for _ in range(_K):
        t = jnp.min(work, axis=1, keepdims=True)
        work = jnp.where(work <= t, jnp.inf, work)
    mask = (dis <= t).astype(jnp.float32)
    cnt = jnp.sum(mask, axis=1, keepdims=True)
    theta = _dot(xf, w_ref[...], ((1,), (0,))) + b_ref[...]
    edge_ref[0] = _dot(mask, theta, ((1,), (0,))) / cnt
    t_ref[0] = t


def _node_kernel(xr_ref, xf_ref, sqc_ref, sqr_ref, t_ref, edge_ref, out_ref):
    xr = xr_ref[0]
    xf = xf_ref[0]
    dis = _dist_tile(xr, xf, sqc_ref[0], sqr_ref[0])
    mask = (dis <= t_ref[0]).astype(jnp.float32)
    dv = jnp.sum(mask, axis=1, keepdims=True)
    out = _dot(mask, edge_ref[0], ((1,), (0,))) / dv
    out_ref[0] = jnp.where(out >= 0, out, _NEG_SLOPE * out)


def _relate_kernel(c0_ref, c1_ref, x1_ref, sqc_ref, sqr_ref, w_ref, b_ref,
                   out_ref):
    dis = _dist_tile(c0_ref[0], c1_ref[0], sqc_ref[0], sqr_ref[0])
    m = jnp.min(dis, axis=1, keepdims=True)
    one = (dis == m).astype(jnp.float32)
    cnt = jnp.sum(one, axis=1, keepdims=True)
    e1 = _dot(x1_ref[0], w_ref[...], ((1,), (0,))) + b_ref[...]
    out_ref[0] = _dot(one, e1, ((1,), (0,))) / cnt


def _pool_kernel(x_ref, w_ref, b_ref, out_ref):
    pool = jnp.mean(x_ref[0], axis=0, keepdims=True)
    out_ref[0] = _dot(pool, w_ref[...], ((1,), (0,))) + b_ref[...]


def _edge_pass(x, sq, w, b):
    B, N, C = x.shape
    Co = w.shape[1]
    nt = N // _BM
    return pl.pallas_call(
        _edge_kernel,
        grid=(B, nt),
        in_specs=[
            pl.BlockSpec((1, _BM, C), lambda b_, i: (b_, i, 0)),
            pl.BlockSpec((1, N, C), lambda b_, i: (b_, 0, 0)),
            pl.BlockSpec((1, _BM, 1), lambda b_, i: (b_, i, 0)),
            pl.BlockSpec((1, 1, N), lambda b_, i: (b_, 0, 0)),
            pl.BlockSpec((C, Co), lambda b_, i: (0, 0)),
            pl.BlockSpec((1, Co), lambda b_, i: (0, 0)),
        ],
        out_specs=[
            pl.BlockSpec((1, _BM, Co), lambda b_, i: (b_, i, 0)),
            pl.BlockSpec((1, _BM, 1), lambda b_, i: (b_, i, 0)),
        ],
        out_shape=[
            jax.ShapeDtypeStruct((B, N, Co), jnp.float32),
            jax.ShapeDtypeStruct((B, N, 1), jnp.float32),
        ],
    )(x, x, sq.reshape(B, N, 1), sq.reshape(B, 1, N), w, b)


def _node_pass(x, sq, t_row, edge):
    B, N, C = x.shape
    Co = edge.shape[2]
    nt = N // _BM
    return pl.pallas_call(
        _node_kernel,
        grid=(B, nt),
        in_specs=[
            pl.BlockSpec((1, _BM, C), lambda b_, i: (b_, i, 0)),
            pl.BlockSpec((1, N, C), lambda b_, i: (b_, 0, 0)),
            pl.BlockSpec((1, _BM, 1), lambda b_, i: (b_, i, 0)),
            pl.BlockSpec((1, 1, N), lambda b_, i: (b_, 0, 0)),
            pl.BlockSpec((1, 1, N), lambda b_, i: (b_, 0, 0)),
            pl.BlockSpec((1, N, Co), lambda b_, i: (b_, 0, 0)),
        ],
        out_specs=pl.BlockSpec((1, _BM, Co), lambda b_, i: (b_, i, 0)),
        out_shape=jax.ShapeDtypeStruct((B, N, Co), jnp.float32),
    )(x, x, sq.reshape(B, N, 1), sq.reshape(B, 1, N), t_row, edge)


def _relate_pass(c0, c1, x1, sq0, sq1, w, b):
    B, N0, dc = c0.shape
    N1 = c1.shape[1]
    d1, Co = w.shape
    nt = N0 // _BM
    return pl.pallas_call(
        _relate_kernel,
        grid=(B, nt),
        in_specs=[
            pl.BlockSpec((1, _BM, dc), lambda b_, i: (b_, i, 0)),
            pl.BlockSpec((1, N1, dc), lambda b_, i: (b_, 0, 0)),
            pl.BlockSpec((1, N1, d1), lambda b_, i: (b_, 0, 0)),
            pl.BlockSpec((1, _BM, 1), lambda b_, i: (b_, i, 0)),
            pl.BlockSpec((1, 1, N1), lambda b_, i: (b_, 0, 0)),
            pl.BlockSpec((d1, Co), lambda b_, i: (0, 0)),
            pl.BlockSpec((1, Co), lambda b_, i: (0, 0)),
        ],
        out_specs=pl.BlockSpec((1, _BM, Co), lambda b_, i: (b_, i, 0)),
        out_shape=jax.ShapeDtypeStruct((B, N0, Co), jnp.float32),
    )(c0, c1, x1, sq0.reshape(B, N0, 1), sq1.reshape(B, 1, N1), w, b)


def _pool_pass(x, w, b):
    B, N, C = x.shape
    Co = w.shape[1]
    return pl.pallas_call(
        _pool_kernel,
        grid=(B,),
        in_specs=[
            pl.BlockSpec((1, N, C), lambda b_: (b_, 0, 0)),
            pl.BlockSpec((C, Co), lambda b_: (0, 0)),
            pl.BlockSpec((1, Co), lambda b_: (0, 0)),
        ],
        out_specs=pl.BlockSpec((1, 1, Co), lambda b_: (b_, 0, 0)),
        out_shape=jax.ShapeDtypeStruct((B, 1, Co), jnp.float32),
    )(x, w, b).reshape(B, Co)


def kernel(x0, x1, c0, c1, W_hg0, b_hg0, W_fc1, b_fc1, W_hg1, b_hg1,
           W_last, b_last):
    B, N0, _ = x0.shape
    sq0 = jnp.sum(x0 * x0, axis=2)
    edge0, t0 = _edge_pass(x0, sq0, W_hg0, b_hg0.reshape(1, -1))
    h = _node_pass(x0, sq0, t0.reshape(B, 1, N0), edge0)
    sqc0 = jnp.sum(c0 * c0, axis=2)
    sqc1 = jnp.sum(c1 * c1, axis=2)
    rel = _relate_pass(c0, c1, x1, sqc0, sqc1, W_fc1, b_fc1.reshape(1, -1))
    hidden = jnp.concatenate([h, rel], axis=-1)
    sqh = jnp.sum(hidden * hidden, axis=2)
    edge1, t1 = _edge_pass(hidden, sqh, W_hg1, b_hg1.reshape(1, -1))
    g = _node_pass(hidden, sqh, t1.reshape(B, 1, N0), edge1)
    return _pool_pass(g, W_last, b_last.reshape(1, -1))


# speedup vs baseline: 19.1275x; 1.0897x over previous
import jax
import jax.numpy as jnp
from jax.experimental import pallas as pl
from jax.experimental.pallas import tpu as pltpu

_K = 16
_NEG_SLOPE = 0.01
_BM = 256


def _dot(a, b, dims):
    return jax.lax.dot_general(a, b, (dims, ((), ())),
                               preferred_element_type=jnp.float32)


def _dist_tile(xr, xf, sq_col, sq_row):
    inner = _dot(xr, xf, ((1,), (1,)))
    return (-2.0 * inner + sq_col) + sq_row


def _hyconv_kernel(xr_ref, xf_ref, sqc_ref, sqr_ref, w_ref, b_ref,
                   out_ref, acc_ref):
    i = pl.program_id(1)
    nt = pl.num_programs(1)
    xr = xr_ref[0]
    xf = xf_ref[0]
    co = w_ref.shape[1]
    dis = _dist_tile(xr, xf, sqc_ref[0], sqr_ref[0])
    work = dis
    t = None
    for _ in range(_K):
        t = jnp.min(work, axis=1, keepdims=True)
        work = jnp.where(work <= t, jnp.inf, work)
    mask = (dis <= t).astype(jnp.float32)
    cnt = jnp.sum(mask, axis=1, keepdims=True)
    theta = _dot(xf, w_ref[...], ((1,), (0,))) + b_ref[...]
    edge = _dot(mask, theta, ((1,), (0,))) / cnt
    edge1 = jnp.concatenate(
        [edge, jnp.ones((edge.shape[0], 1), jnp.float32)], axis=1)
    contrib = _dot(mask, edge1, ((0,), (0,)))

    @pl.when(i == 0)
    def _():
        acc_ref[...] = contrib

    @pl.when(i > 0)
    def _():
        acc_ref[...] += contrib

    @pl.when(i == nt - 1)
    def _():
        acc = acc_ref[...]
        out = acc[:, :co] / acc[:, co:co + 1]
        out_ref[0] = jnp.where(out >= 0, out, _NEG_SLOPE * out)


def _relate_kernel(c0_ref, c1_ref, x1_ref, sqc_ref, sqr_ref, w_ref, b_ref,
                   out_ref):
    dis = _dist_tile(c0_ref[0], c1_ref[0], sqc_ref[0], sqr_ref[0])
    m = jnp.min(dis, axis=1, keepdims=True)
    one = (dis == m).astype(jnp.float32)
    cnt = jnp.sum(one, axis=1, keepdims=True)
    e1 = _dot(x1_ref[0], w_ref[...], ((1,), (0,))) + b_ref[...]
    out_ref[0] = _dot(one, e1, ((1,), (0,))) / cnt


def _pool_kernel(x_ref, w_ref, b_ref, out_ref):
    pool = jnp.mean(x_ref[0], axis=0, keepdims=True)
    out_ref[0] = _dot(pool, w_ref[...], ((1,), (0,))) + b_ref[...]


def _hyconv_pass(x, sq, w, b):
    B, N, C = x.shape
    Co = w.shape[1]
    nt = N // _BM
    return pl.pallas_call(
        _hyconv_kernel,
        grid=(B, nt),
        in_specs=[
            pl.BlockSpec((1, _BM, C), lambda b_, i: (b_, i, 0)),
            pl.BlockSpec((1, N, C), lambda b_, i: (b_, 0, 0)),
            pl.BlockSpec((1, _BM, 1), lambda b_, i: (b_, i, 0)),
            pl.BlockSpec((1, 1, N), lambda b_, i: (b_, 0, 0)),
            pl.BlockSpec((C, Co), lambda b_, i: (0, 0)),
            pl.BlockSpec((1, Co), lambda b_, i: (0, 0)),
        ],
        out_specs=pl.BlockSpec((1, N, Co), lambda b_, i: (b_, 0, 0)),
        out_shape=jax.ShapeDtypeStruct((B, N, Co), jnp.float32),
        scratch_shapes=[pltpu.VMEM((N, Co + 1), jnp.float32)],
    )(x, x, sq.reshape(B, N, 1), sq.reshape(B, 1, N), w, b)


def _relate_pass(c0, c1, x1, sq0, sq1, w, b):
    B, N0, dc = c0.shape
    N1 = c1.shape[1]
    d1, Co = w.shape
    nt = N0 // _BM
    return pl.pallas_call(
        _relate_kernel,
        grid=(B, nt),
        in_specs=[
            pl.BlockSpec((1, _BM, dc), lambda b_, i: (b_, i, 0)),
            pl.BlockSpec((1, N1, dc), lambda b_, i: (b_, 0, 0)),
            pl.BlockSpec((1, N1, d1), lambda b_, i: (b_, 0, 0)),
            pl.BlockSpec((1, _BM, 1), lambda b_, i: (b_, i, 0)),
            pl.BlockSpec((1, 1, N1), lambda b_, i: (b_, 0, 0)),
            pl.BlockSpec((d1, Co), lambda b_, i: (0, 0)),
            pl.BlockSpec((1, Co), lambda b_, i: (0, 0)),
        ],
        out_specs=pl.BlockSpec((1, _BM, Co), lambda b_, i: (b_, i, 0)),
        out_shape=jax.ShapeDtypeStruct((B, N0, Co), jnp.float32),
    )(c0, c1, x1, sq0.reshape(B, N0, 1), sq1.reshape(B, 1, N1), w, b)


def _pool_pass(x, w, b):
    B, N, C = x.shape
    Co = w.shape[1]
    return pl.pallas_call(
        _pool_kernel,
        grid=(B,),
        in_specs=[
            pl.BlockSpec((1, N, C), lambda b_: (b_, 0, 0)),
            pl.BlockSpec((C, Co), lambda b_: (0, 0)),
            pl.BlockSpec((1, Co), lambda b_: (0, 0)),
        ],
        out_specs=pl.BlockSpec((1, 1, Co), lambda b_: (b_, 0, 0)),
        out_shape=jax.ShapeDtypeStruct((B, 1, Co), jnp.float32),
    )(x, w, b).reshape(B, Co)


def kernel(x0, x1, c0, c1, W_hg0, b_hg0, W_fc1, b_fc1, W_hg1, b_hg1,
           W_last, b_last):
    sq0 = jnp.sum(x0 * x0, axis=2)
    h = _hyconv_pass(x0, sq0, W_hg0, b_hg0.reshape(1, -1))
    sqc0 = jnp.sum(c0 * c0, axis=2)
    sqc1 = jnp.sum(c1 * c1, axis=2)
    rel = _relate_pass(c0, c1, x1, sqc0, sqc1, W_fc1, b_fc1.reshape(1, -1))
    hidden = jnp.concatenate([h, rel], axis=-1)
    sqh = jnp.sum(hidden * hidden, axis=2)
    g = _hyconv_pass(hidden, sqh, W_hg1, b_hg1.reshape(1, -1))
    return _pool_pass(g, W_last, b_last.reshape(1, -1))


# speedup vs baseline: 22.0391x; 1.1522x over previous
import jax
import jax.numpy as jnp
from jax.experimental import pallas as pl
from jax.experimental.pallas import tpu as pltpu

_K = 16
_NEG_SLOPE = 0.01
_BM = 512
_SUB = 256


def _dot(a, b, dims):
    return jax.lax.dot_general(a, b, (dims, ((), ())),
                               preferred_element_type=jnp.float32)


def _dist_tile(xr, xf, sq_col, sq_row):
    inner = _dot(xr, xf, ((1,), (1,)))
    return (-2.0 * inner + sq_col) + sq_row


def _hyconv_kernel(xr_ref, xf_ref, sqc_ref, sqr_ref, w_ref, b_ref,
                   out_ref, acc_ref):
    i = pl.program_id(1)
    nt = pl.num_programs(1)
    xr = xr_ref[0]
    xf = xf_ref[0]
    co = w_ref.shape[1]
    theta = _dot(xf, w_ref[...], ((1,), (0,))) + b_ref[...]
    contribs = []
    for h in range(_BM // _SUB):
        s = slice(h * _SUB, (h + 1) * _SUB)
        dis = _dist_tile(xr[s], xf, sqc_ref[0][s], sqr_ref[0])
        work = dis
        t = None
        for _ in range(_K):
            t = jnp.min(work, axis=1, keepdims=True)
            work = jnp.where(work <= t, jnp.inf, work)
        mask = (dis <= t).astype(jnp.float32)
        cnt = jnp.sum(mask, axis=1, keepdims=True)
        edge = _dot(mask, theta, ((1,), (0,))) / cnt
        edge1 = jnp.concatenate(
            [edge, jnp.ones((edge.shape[0], 1), jnp.float32)], axis=1)
        contribs.append(_dot(mask, edge1, ((0,), (0,))))
    contrib = sum(contribs)

    @pl.when(i == 0)
    def _():
        acc_ref[...] = contrib

    @pl.when(i > 0)
    def _():
        acc_ref[...] += contrib

    @pl.when(i == nt - 1)
    def _():
        acc = acc_ref[...]
        out = acc[:, :co] / acc[:, co:co + 1]
        out_ref[0] = jnp.where(out >= 0, out, _NEG_SLOPE * out)


def _relate_kernel(c0_ref, c1_ref, x1_ref, sqc_ref, sqr_ref, w_ref, b_ref,
                   out_ref):
    dis = _dist_tile(c0_ref[0], c1_ref[0], sqc_ref[0], sqr_ref[0])
    m = jnp.min(dis, axis=1, keepdims=True)
    one = (dis == m).astype(jnp.float32)
    cnt = jnp.sum(one, axis=1, keepdims=True)
    e1 = _dot(x1_ref[0], w_ref[...], ((1,), (0,))) + b_ref[...]
    out_ref[0] = _dot(one, e1, ((1,), (0,))) / cnt


def _pool_kernel(x_ref, w_ref, b_ref, out_ref):
    pool = jnp.mean(x_ref[0], axis=0, keepdims=True)
    out_ref[0] = _dot(pool, w_ref[...], ((1,), (0,))) + b_ref[...]


def _hyconv_pass(x, sq, w, b):
    B, N, C = x.shape
    Co = w.shape[1]
    nt = N // _BM
    return pl.pallas_call(
        _hyconv_kernel,
        grid=(B, nt),
        in_specs=[
            pl.BlockSpec((1, _BM, C), lambda b_, i: (b_, i, 0)),
            pl.BlockSpec((1, N, C), lambda b_, i: (b_, 0, 0)),
            pl.BlockSpec((1, _BM, 1), lambda b_, i: (b_, i, 0)),
            pl.BlockSpec((1, 1, N), lambda b_, i: (b_, 0, 0)),
            pl.BlockSpec((C, Co), lambda b_, i: (0, 0)),
            pl.BlockSpec((1, Co), lambda b_, i: (0, 0)),
        ],
        out_specs=pl.BlockSpec((1, N, Co), lambda b_, i: (b_, 0, 0)),
        out_shape=jax.ShapeDtypeStruct((B, N, Co), jnp.float32),
        scratch_shapes=[pltpu.VMEM((N, Co + 1), jnp.float32)],
    )(x, x, sq.reshape(B, N, 1), sq.reshape(B, 1, N), w, b)


def _relate_pass(c0, c1, x1, sq0, sq1, w, b):
    B, N0, dc = c0.shape
    N1 = c1.shape[1]
    d1, Co = w.shape
    nt = N0 // _BM
    return pl.pallas_call(
        _relate_kernel,
        grid=(B, nt),
        in_specs=[
            pl.BlockSpec((1, _BM, dc), lambda b_, i: (b_, i, 0)),
            pl.BlockSpec((1, N1, dc), lambda b_, i: (b_, 0, 0)),
            pl.BlockSpec((1, N1, d1), lambda b_, i: (b_, 0, 0)),
            pl.BlockSpec((1, _BM, 1), lambda b_, i: (b_, i, 0)),
            pl.BlockSpec((1, 1, N1), lambda b_, i: (b_, 0, 0)),
            pl.BlockSpec((d1, Co), lambda b_, i: (0, 0)),
            pl.BlockSpec((1, Co), lambda b_, i: (0, 0)),
        ],
        out_specs=pl.BlockSpec((1, _BM, Co), lambda b_, i: (b_, i, 0)),
        out_shape=jax.ShapeDtypeStruct((B, N0, Co), jnp.float32),
    )(c0, c1, x1, sq0.reshape(B, N0, 1), sq1.reshape(B, 1, N1), w, b)


def _pool_pass(x, w, b):
    B, N, C = x.shape
    Co = w.shape[1]
    return pl.pallas_call(
        _pool_kernel,
        grid=(B,),
        in_specs=[
            pl.BlockSpec((1, N, C), lambda b_: (b_, 0, 0)),
            pl.BlockSpec((C, Co), lambda b_: (0, 0)),
            pl.BlockSpec((1, Co), lambda b_: (0, 0)),
        ],
        out_specs=pl.BlockSpec((1, 1, Co), lambda b_: (b_, 0, 0)),
        out_shape=jax.ShapeDtypeStruct((B, 1, Co), jnp.float32),
    )(x, w, b).reshape(B, Co)


def kernel(x0, x1, c0, c1, W_hg0, b_hg0, W_fc1, b_fc1, W_hg1, b_hg1,
           W_last, b_last):
    sq0 = jnp.sum(x0 * x0, axis=2)
    h = _hyconv_pass(x0, sq0, W_hg0, b_hg0.reshape(1, -1))
    sqc0 = jnp.sum(c0 * c0, axis=2)
    sqc1 = jnp.sum(c1 * c1, axis=2)
    rel = _relate_pass(c0, c1, x1, sqc0, sqc1, W_fc1, b_fc1.reshape(1, -1))
    hidden = jnp.concatenate([h, rel], axis=-1)
    sqh = jnp.sum(hidden * hidden, axis=2)
    g = _hyconv_pass(hidden, sqh, W_hg1, b_hg1.reshape(1, -1))
    return _pool_pass(g, W_last, b_last.reshape(1, -1))


# speedup vs baseline: 22.1960x; 1.0071x over previous
import jax
import jax.numpy as jnp
from jax.experimental import pallas as pl
from jax.experimental.pallas import tpu as pltpu

_K = 16
_NEG_SLOPE = 0.01
_BM = 512
_SUB = 256


def _dot(a, b, dims):
    return jax.lax.dot_general(a, b, (dims, ((), ())),
                               preferred_element_type=jnp.float32)


def _dist_tile(xr, xf, sq_col, sq_row):
    inner = _dot(xr, xf, ((1,), (1,)))
    return (-2.0 * inner + sq_col) + sq_row


def _hyconv_kernel(xr_ref, xf_ref, sqc_ref, sqr_ref, w_ref, b_ref,
                   out_ref, acc_ref):
    i = pl.program_id(1)
    nt = pl.num_programs(1)
    xr = xr_ref[0]
    xf = xf_ref[0]
    co = w_ref.shape[1]
    theta = _dot(xf, w_ref[...], ((1,), (0,))) + b_ref[...]
    contribs = []
    for h in range(_BM // _SUB):
        s = slice(h * _SUB, (h + 1) * _SUB)
        dis = _dist_tile(xr[s], xf, sqc_ref[0][s], sqr_ref[0])
        t = jnp.min(dis, axis=1, keepdims=True)
        for _ in range(_K - 1):
            t = jnp.min(jnp.where(dis <= t, jnp.inf, dis),
                        axis=1, keepdims=True)
        mask = (dis <= t).astype(jnp.float32)
        cnt = jnp.sum(mask, axis=1, keepdims=True)
        edge = _dot(mask, theta, ((1,), (0,))) / cnt
        edge1 = jnp.concatenate(
            [edge, jnp.ones((edge.shape[0], 1), jnp.float32)], axis=1)
        contribs.append(_dot(mask, edge1, ((0,), (0,))))
    contrib = sum(contribs)

    @pl.when(i == 0)
    def _():
        acc_ref[...] = contrib

    @pl.when(i > 0)
    def _():
        acc_ref[...] += contrib

    @pl.when(i == nt - 1)
    def _():
        acc = acc_ref[...]
        out = acc[:, :co] / acc[:, co:co + 1]
        out_ref[0] = jnp.where(out >= 0, out, _NEG_SLOPE * out)


def _relate_kernel(c0_ref, c1_ref, x1_ref, sqc_ref, sqr_ref, w_ref, b_ref,
                   out_ref):
    dis = _dist_tile(c0_ref[0], c1_ref[0], sqc_ref[0], sqr_ref[0])
    m = jnp.min(dis, axis=1, keepdims=True)
    one = (dis == m).astype(jnp.float32)
    cnt = jnp.sum(one, axis=1, keepdims=True)
    e1 = _dot(x1_ref[0], w_ref[...], ((1,), (0,))) + b_ref[...]
    out_ref[0] = _dot(one, e1, ((1,), (0,))) / cnt


def _pool_kernel(x_ref, w_ref, b_ref, out_ref):
    pool = jnp.mean(x_ref[0], axis=0, keepdims=True)
    out_ref[0] = _dot(pool, w_ref[...], ((1,), (0,))) + b_ref[...]


def _hyconv_pass(x, sq, w, b):
    B, N, C = x.shape
    Co = w.shape[1]
    nt = N // _BM
    return pl.pallas_call(
        _hyconv_kernel,
        grid=(B, nt),
        in_specs=[
            pl.BlockSpec((1, _BM, C), lambda b_, i: (b_, i, 0)),
            pl.BlockSpec((1, N, C), lambda b_, i: (b_, 0, 0)),
            pl.BlockSpec((1, _BM, 1), lambda b_, i: (b_, i, 0)),
            pl.BlockSpec((1, 1, N), lambda b_, i: (b_, 0, 0)),
            pl.BlockSpec((C, Co), lambda b_, i: (0, 0)),
            pl.BlockSpec((1, Co), lambda b_, i: (0, 0)),
        ],
        out_specs=pl.BlockSpec((1, N, Co), lambda b_, i: (b_, 0, 0)),
        out_shape=jax.ShapeDtypeStruct((B, N, Co), jnp.float32),
        scratch_shapes=[pltpu.VMEM((N, Co + 1), jnp.float32)],
    )(x, x, sq.reshape(B, N, 1), sq.reshape(B, 1, N), w, b)


def _relate_pass(c0, c1, x1, sq0, sq1, w, b):
    B, N0, dc = c0.shape
    N1 = c1.shape[1]
    d1, Co = w.shape
    nt = N0 // _BM
    return pl.pallas_call(
        _relate_kernel,
        grid=(B, nt),
        in_specs=[
            pl.BlockSpec((1, _BM, dc), lambda b_, i: (b_, i, 0)),
            pl.BlockSpec((1, N1, dc), lambda b_, i: (b_, 0, 0)),
            pl.BlockSpec((1, N1, d1), lambda b_, i: (b_, 0, 0)),
            pl.BlockSpec((1, _BM, 1), lambda b_, i: (b_, i, 0)),
            pl.BlockSpec((1, 1, N1), lambda b_, i: (b_, 0, 0)),
            pl.BlockSpec((d1, Co), lambda b_, i: (0, 0)),
            pl.BlockSpec((1, Co), lambda b_, i: (0, 0)),
        ],
        out_specs=pl.BlockSpec((1, _BM, Co), lambda b_, i: (b_, i, 0)),
        out_shape=jax.ShapeDtypeStruct((B, N0, Co), jnp.float32),
    )(c0, c1, x1, sq0.reshape(B, N0, 1), sq1.reshape(B, 1, N1), w, b)


def _pool_pass(x, w, b):
    B, N, C = x.shape
    Co = w.shape[1]
    return pl.pallas_call(
        _pool_kernel,
        grid=(B,),
        in_specs=[
            pl.BlockSpec((1, N, C), lambda b_: (b_, 0, 0)),
            pl.BlockSpec((C, Co), lambda b_: (0, 0)),
            pl.BlockSpec((1, Co), lambda b_: (0, 0)),
        ],
        out_specs=pl.BlockSpec((1, 1, Co), lambda b_: (b_, 0, 0)),
        out_shape=jax.ShapeDtypeStruct((B, 1, Co), jnp.float32),
    )(x, w, b).reshape(B, Co)


def kernel(x0, x1, c0, c1, W_hg0, b_hg0, W_fc1, b_fc1, W_hg1, b_hg1,
           W_last, b_last):
    sq0 = jnp.sum(x0 * x0, axis=2)
    h = _hyconv_pass(x0, sq0, W_hg0, b_hg0.reshape(1, -1))
    sqc0 = jnp.sum(c0 * c0, axis=2)
    sqc1 = jnp.sum(c1 * c1, axis=2)
    rel = _relate_pass(c0, c1, x1, sqc0, sqc1, W_fc1, b_fc1.reshape(1, -1))
    hidden = jnp.concatenate([h, rel], axis=-1)
    sqh = jnp.sum(hidden * hidden, axis=2)
    g = _hyconv_pass(hidden, sqh, W_hg1, b_hg1.reshape(1, -1))
    return _pool_pass(g, W_last, b_last.reshape(1, -1))
